# baseline (speedup 1.0000x reference)
import functools

import jax
import jax.numpy as jnp
from jax import lax
from jax.experimental import pallas as pl
from jax.experimental.pallas import tpu as pltpu
from jax.experimental.pallas import tpu_sc as plsc

N = 10000
E = 320000
D = 128
H = 128
G = 16
OUT = 64

NP = 10016
NC = 2
NS = 16
NW = NC * NS
EPT = E // NW
CHUNK = 64
NSLOT = 2
NCH = 4 * (-(-EPT // (CHUNK * 4)))
NJ = NCH // 4
EPT_PAD = NCH * CHUNK
ROWS_PT = NP // NS
NEG = -1e30


def _prep_body(x_ref, w_ref, as_ref, ad_ref, xpw_ref, asrc_ref, adst_ref,
               init_ref, exs_ref):
    i = pl.program_id(0)
    br = x_ref.shape[0]
    xp = jnp.dot(x_ref[...], w_ref[...], preferred_element_type=jnp.float32)
    xpw_ref[...] = xp

    rows = i * br + lax.broadcasted_iota(jnp.int32, (br,), 0)
    valid = rows < N
    asrc = jnp.sum(xp * as_ref[...], axis=1)
    adst = jnp.sum(xp * ad_ref[...], axis=1)
    asrc = jnp.where(valid, asrc, NEG)
    adst = jnp.where(valid, adst, NEG)
    asrc_ref[...] = asrc[:, None]
    adst_ref[...] = adst[:, None]

    a = asrc + adst
    a = jnp.where(a >= 0, a, 0.2 * a)
    ex_self = jnp.where(valid, jnp.exp(a), 1.0)
    exs_ref[...] = ex_self[:, None]
    init0 = xp * ex_self[:, None]
    init_ref[...] = jnp.stack([init0, jnp.zeros_like(init0)], axis=0)


def _prep(x_p, W, a_s, a_d):
    br = NP // 4
    return pl.pallas_call(
        _prep_body,
        grid=(4,),
        in_specs=[
            pl.BlockSpec((br, D), lambda i: (i, 0)),
            pl.BlockSpec((D, H), lambda i: (0, 0)),
            pl.BlockSpec((1, H), lambda i: (0, 0)),
            pl.BlockSpec((1, H), lambda i: (0, 0)),
        ],
        out_specs=[
            pl.BlockSpec((br, H), lambda i: (i, 0)),
            pl.BlockSpec((br, 1), lambda i: (i, 0)),
            pl.BlockSpec((br, 1), lambda i: (i, 0)),
            pl.BlockSpec((2, br, H), lambda i: (0, i, 0)),
            pl.BlockSpec((br, 1), lambda i: (i, 0)),
        ],
        out_shape=[
            jax.ShapeDtypeStruct((NP, H), jnp.float32),
            jax.ShapeDtypeStruct((NP, 1), jnp.float32),
            jax.ShapeDtypeStruct((NP, 1), jnp.float32),
            jax.ShapeDtypeStruct((2, NP, H), jnp.float32),
            jax.ShapeDtypeStruct((NP, 1), jnp.float32),
        ],
    )(x_p, W, a_s.reshape(1, H), a_d.reshape(1, H))


def _edge_body(xpw_hbm, av_hbm, eii_hbm, init_hbm,
               out_hbm, den_hbm, av_v, eii_v, ex_v, rows_v, den_v, acc_sh,
               isem, gsem, ssem):
    c = lax.axis_index("c")
    s = lax.axis_index("s")
    wid = c * NS + s

    r0 = s * ROWS_PT
    pltpu.sync_copy(init_hbm.at[c, pl.ds(r0, ROWS_PT)],
                    acc_sh.at[pl.ds(r0, ROWS_PT)])
    pltpu.sync_copy(av_hbm, av_v)

    def zf(i, _):
        den_v[pl.ds(i * 16, 16)] = jnp.zeros((16,), jnp.float32)
        return 0
    lax.fori_loop(0, NP // 16, zf, 0)
    plsc.subcore_barrier()

    row0 = wid * NCH

    def idx_start(j, b):
        return pltpu.make_async_copy(
            eii_hbm.at[pl.ds(row0 + j * 4 + 2 * b, 2)], eii_v.at[b],
            isem.at[b])

    def do_pair(b):
        g0 = pltpu.async_copy(xpw_hbm.at[eii_v.at[b, 0, 0]], rows_v.at[0],
                              gsem.at[0])
        g1 = pltpu.async_copy(xpw_hbm.at[eii_v.at[b, 1, 0]], rows_v.at[1],
                              gsem.at[1])
        for sl in range(2):
            for i in range(CHUNK // 16):
                d16 = pl.ds(i * 16, 16)
                dsti = eii_v[b, sl, 1, d16]
                a = (plsc.load_gather(av_v, [eii_v[b, sl, 0, d16]])
                     + plsc.load_gather(av_v, [dsti + NP]))
                a = jnp.where(a >= 0, a, jnp.float32(0.2) * a)
                ex = jnp.exp(a)
                ex_v[sl, d16] = ex
                plsc.addupdate_scatter(den_v, [dsti], ex)

        def scale_slot(sl):
            def scale(g, _):
                ev = ex_v[sl, pl.ds(g * 16, 16)]
                for k in range(16):
                    sc = ev[k]
                    r = g * 16 + k
                    for jj in range(H // 16):
                        cs = pl.ds(jj * 16, 16)
                        rows_v[sl, r, cs] = rows_v[sl, r, cs] * sc
                return 0
            lax.fori_loop(0, CHUNK // 16, scale, 0)

        g0.wait()
        scale_slot(0)
        s0 = pltpu.async_copy(rows_v.at[0], acc_sh.at[eii_v.at[b, 0, 1]],
                              ssem.at[0], add=True)
        g1.wait()
        scale_slot(1)
        s1 = pltpu.async_copy(rows_v.at[1], acc_sh.at[eii_v.at[b, 1, 1]],
                              ssem.at[1], add=True)
        s0.wait()
        s1.wait()

    idx_start(0, 0).start()
    idx_start(0, 1).start()

    def gen(j, _):
        idx_start(j, 0).wait()
        do_pair(0)

        @pl.when(j < NJ - 1)
        def _():
            idx_start(j + 1, 0).start()
        idx_start(j, 1).wait()
        do_pair(1)

        @pl.when(j < NJ - 1)
        def _():
            idx_start(j + 1, 1).start()
        return 0

    lax.fori_loop(0, NJ, gen, 0)
    pltpu.sync_copy(den_v, den_hbm.at[c, s])
    plsc.subcore_barrier()
    pltpu.sync_copy(acc_sh.at[pl.ds(r0, ROWS_PT)],
                    out_hbm.at[c, pl.ds(r0, ROWS_PT)])


def _edge_phase(xpw, av, eii, init):
    mesh = plsc.VectorSubcoreMesh(core_axis_name="c", subcore_axis_name="s",
                                  num_cores=NC, num_subcores=NS)
    f = pl.kernel(
        _edge_body,
        out_type=[jax.ShapeDtypeStruct((2, NP, H), jnp.float32),
                  jax.ShapeDtypeStruct((NC, NS, NP), jnp.float32)],
        mesh=mesh,
        scratch_types=[
            pltpu.VMEM((2 * NP,), jnp.float32),
            pltpu.VMEM((NSLOT, 2, 2, CHUNK), jnp.int32),
            pltpu.VMEM((NSLOT, CHUNK), jnp.float32),
            pltpu.VMEM((NSLOT, CHUNK, H), jnp.float32),
            pltpu.VMEM((NP,), jnp.float32),
            pltpu.VMEM_SHARED((NP, H), jnp.float32),
            pltpu.SemaphoreType.DMA((NSLOT,)),
            pltpu.SemaphoreType.DMA((NSLOT,)),
            pltpu.SemaphoreType.DMA((NSLOT,)),
        ],
        compiler_params=pltpu.CompilerParams(needs_layout_passes=False,
                                             use_tc_tiling_on_sc=False),
    )
    return f(xpw, av, eii, init)


def _finish_body(part_ref, dens_ref, exs_ref, b_ref, batch_ref, lw_ref,
                 lb_ref, out_ref):
    numer = part_ref[0] + part_ref[1]
    denom = jnp.sum(dens_ref[...], axis=0)[:, None] + exs_ref[...]
    h = jnp.maximum(numer / denom + b_ref[...], 0.0)
    gids = lax.broadcasted_iota(jnp.int32, (G, NP), 0)
    oh = (gids == batch_ref[...]).astype(jnp.float32)
    sums = jnp.dot(oh, h, preferred_element_type=jnp.float32)
    cnt = jnp.sum(oh, axis=1, keepdims=True)
    pooled = sums / jnp.maximum(cnt, 1.0)
    out_ref[...] = (jnp.dot(pooled, lw_ref[...],
                            preferred_element_type=jnp.float32) + lb_ref[...])


def _finish(part, dens, exs, b, batch_p, linW, linb):
    return pl.pallas_call(
        _finish_body,
        out_shape=jax.ShapeDtypeStruct((G, OUT), jnp.float32),
    )(part, dens, exs, b.reshape(1, H), batch_p, linW, linb.reshape(1, OUT))


def kernel(x, edge_index, batch, W1, a_s1, a_d1, b1, W2, a_s2, a_d2, b2,
           linW, linb):
    del W1, a_s1, a_d1, b1
    x_p = jnp.concatenate([x, jnp.zeros((NP - N, D), jnp.float32)], axis=0)
    xpw, asrc, adst, init, exs = _prep(x_p, W2, a_s2, a_d2)
    asrc = asrc.reshape(NP)
    adst = adst.reshape(NP)

    pad = jnp.full((NW, EPT_PAD - EPT), N, jnp.int32)
    src_p = jnp.concatenate([edge_index[0].reshape(NW, EPT), pad],
                            axis=1).reshape(NW * NCH, CHUNK)
    dst_p = jnp.concatenate([edge_index[1].reshape(NW, EPT), pad],
                            axis=1).reshape(NW * NCH, CHUNK)
    eii = jnp.stack([src_p, dst_p], axis=1)
    av = jnp.concatenate([asrc, adst])

    part, dens = _edge_phase(xpw, av, eii, init)

    batch_p = jnp.concatenate(
        [batch, jnp.full((NP - N,), G, jnp.int32)]).reshape(1, NP)
    return _finish(part, dens.reshape(NW, NP), exs, b2, batch_p, linW, linb)


# speedup vs baseline: 1.1922x; 1.1922x over previous
import functools

import jax
import jax.numpy as jnp
from jax import lax
from jax.experimental import pallas as pl
from jax.experimental.pallas import tpu as pltpu
from jax.experimental.pallas import tpu_sc as plsc

N = 10000
E = 320000
D = 128
H = 128
G = 16
OUT = 64

NP = 10016
NC = 2
NS = 16
NW = NC * NS
EPT = E // NW
CHUNK = 32
NSLOT = 2
NCH = 4 * (-(-EPT // (CHUNK * 4)))
NJ = NCH // 4
EPT_PAD = NCH * CHUNK
ROWS_PT = NP // NS
NEG = -1e30


def _prep_body(x_ref, w_ref, as_ref, ad_ref, xpw_ref, asrc_ref, adst_ref,
               init_ref, exs_ref):
    i = pl.program_id(0)
    br = x_ref.shape[0]
    xp = jnp.dot(x_ref[...], w_ref[...], preferred_element_type=jnp.float32)
    xpw_ref[...] = xp

    rows = i * br + lax.broadcasted_iota(jnp.int32, (br,), 0)
    valid = rows < N
    asrc = jnp.sum(xp * as_ref[...], axis=1)
    adst = jnp.sum(xp * ad_ref[...], axis=1)
    asrc = jnp.where(valid, asrc, NEG)
    adst = jnp.where(valid, adst, NEG)
    asrc_ref[...] = asrc[:, None]
    adst_ref[...] = adst[:, None]

    a = asrc + adst
    a = jnp.where(a >= 0, a, 0.2 * a)
    ex_self = jnp.where(valid, jnp.exp(a), 1.0)
    exs_ref[...] = ex_self[:, None]
    init0 = xp * ex_self[:, None]
    init_ref[...] = jnp.stack([init0, jnp.zeros_like(init0)], axis=0)


def _prep(x_p, W, a_s, a_d):
    br = NP // 4
    return pl.pallas_call(
        _prep_body,
        grid=(4,),
        in_specs=[
            pl.BlockSpec((br, D), lambda i: (i, 0)),
            pl.BlockSpec((D, H), lambda i: (0, 0)),
            pl.BlockSpec((1, H), lambda i: (0, 0)),
            pl.BlockSpec((1, H), lambda i: (0, 0)),
        ],
        out_specs=[
            pl.BlockSpec((br, H), lambda i: (i, 0)),
            pl.BlockSpec((br, 1), lambda i: (i, 0)),
            pl.BlockSpec((br, 1), lambda i: (i, 0)),
            pl.BlockSpec((2, br, H), lambda i: (0, i, 0)),
            pl.BlockSpec((br, 1), lambda i: (i, 0)),
        ],
        out_shape=[
            jax.ShapeDtypeStruct((NP, H), jnp.float32),
            jax.ShapeDtypeStruct((NP, 1), jnp.float32),
            jax.ShapeDtypeStruct((NP, 1), jnp.float32),
            jax.ShapeDtypeStruct((2, NP, H), jnp.float32),
            jax.ShapeDtypeStruct((NP, 1), jnp.float32),
        ],
    )(x_p, W, a_s.reshape(1, H), a_d.reshape(1, H))


def _edge_body(xpw_hbm, av_hbm, eii_hbm, init_hbm,
               out_hbm, den_hbm, av_v, eii_v, ex_v, rows_v, den_v, acc_sh,
               isem, gsem, ssem):
    c = lax.axis_index("c")
    s = lax.axis_index("s")
    wid = c * NS + s

    r0 = s * ROWS_PT
    pltpu.sync_copy(init_hbm.at[c, pl.ds(r0, ROWS_PT)],
                    acc_sh.at[pl.ds(r0, ROWS_PT)])
    pltpu.sync_copy(av_hbm, av_v)

    def zf(i, _):
        den_v[pl.ds(i * 16, 16)] = jnp.zeros((16,), jnp.float32)
        return 0
    lax.fori_loop(0, NP // 16, zf, 0)
    plsc.subcore_barrier()

    row0 = wid * NCH

    def idx_start(j, b):
        return pltpu.make_async_copy(
            eii_hbm.at[pl.ds(row0 + j * 4 + 2 * b, 2)], eii_v.at[b],
            isem.at[b])

    def do_pair(b):
        g0 = pltpu.async_copy(xpw_hbm.at[eii_v.at[b, 0, 0]], rows_v.at[0],
                              gsem.at[0])
        g1 = pltpu.async_copy(xpw_hbm.at[eii_v.at[b, 1, 0]], rows_v.at[1],
                              gsem.at[1])
        for sl in range(2):
            for i in range(CHUNK // 16):
                d16 = pl.ds(i * 16, 16)
                dsti = eii_v[b, sl, 1, d16]
                a = (plsc.load_gather(av_v, [eii_v[b, sl, 0, d16]])
                     + plsc.load_gather(av_v, [dsti + NP]))
                a = jnp.where(a >= 0, a, jnp.float32(0.2) * a)
                ex = jnp.exp(a)
                ex_v[sl, d16] = ex
                plsc.addupdate_scatter(den_v, [dsti], ex)

        def scale_slot(sl):
            def scale(g, _):
                ev = ex_v[sl, pl.ds(g * 16, 16)]
                for k in range(16):
                    sc = ev[k]
                    r = g * 16 + k
                    for jj in range(H // 16):
                        cs = pl.ds(jj * 16, 16)
                        rows_v[sl, r, cs] = rows_v[sl, r, cs] * sc
                return 0
            lax.fori_loop(0, CHUNK // 16, scale, 0)

        g0.wait()
        scale_slot(0)
        s0 = pltpu.async_copy(rows_v.at[0], acc_sh.at[eii_v.at[b, 0, 1]],
                              ssem.at[0], add=True)
        g1.wait()
        scale_slot(1)
        s1 = pltpu.async_copy(rows_v.at[1], acc_sh.at[eii_v.at[b, 1, 1]],
                              ssem.at[1], add=True)
        s0.wait()
        s1.wait()

    idx_start(0, 0).start()
    idx_start(0, 1).start()

    def gen(j, _):
        idx_start(j, 0).wait()
        do_pair(0)

        @pl.when(j < NJ - 1)
        def _():
            idx_start(j + 1, 0).start()
        idx_start(j, 1).wait()
        do_pair(1)

        @pl.when(j < NJ - 1)
        def _():
            idx_start(j + 1, 1).start()
        return 0

    lax.fori_loop(0, NJ, gen, 0)
    pltpu.sync_copy(den_v, den_hbm.at[c, s])
    plsc.subcore_barrier()
    pltpu.sync_copy(acc_sh.at[pl.ds(r0, ROWS_PT)],
                    out_hbm.at[c, pl.ds(r0, ROWS_PT)])


def _edge_phase(xpw, av, eii, init):
    mesh = plsc.VectorSubcoreMesh(core_axis_name="c", subcore_axis_name="s",
                                  num_cores=NC, num_subcores=NS)
    f = pl.kernel(
        _edge_body,
        out_type=[jax.ShapeDtypeStruct((2, NP, H), jnp.float32),
                  jax.ShapeDtypeStruct((NC, NS, NP), jnp.float32)],
        mesh=mesh,
        scratch_types=[
            pltpu.VMEM((2 * NP,), jnp.float32),
            pltpu.VMEM((NSLOT, 2, 2, CHUNK), jnp.int32),
            pltpu.VMEM((NSLOT, CHUNK), jnp.float32),
            pltpu.VMEM((NSLOT, CHUNK, H), jnp.float32),
            pltpu.VMEM((NP,), jnp.float32),
            pltpu.VMEM_SHARED((NP, H), jnp.float32),
            pltpu.SemaphoreType.DMA((NSLOT,)),
            pltpu.SemaphoreType.DMA((NSLOT,)),
            pltpu.SemaphoreType.DMA((NSLOT,)),
        ],
        compiler_params=pltpu.CompilerParams(needs_layout_passes=False,
                                             use_tc_tiling_on_sc=False),
    )
    return f(xpw, av, eii, init)


def _finish_body(part_ref, dens_ref, exs_ref, b_ref, batch_ref, lw_ref,
                 lb_ref, out_ref):
    numer = part_ref[0] + part_ref[1]
    denom = jnp.sum(dens_ref[...], axis=0)[:, None] + exs_ref[...]
    h = jnp.maximum(numer / denom + b_ref[...], 0.0)
    gids = lax.broadcasted_iota(jnp.int32, (G, NP), 0)
    oh = (gids == batch_ref[...]).astype(jnp.float32)
    sums = jnp.dot(oh, h, preferred_element_type=jnp.float32)
    cnt = jnp.sum(oh, axis=1, keepdims=True)
    pooled = sums / jnp.maximum(cnt, 1.0)
    out_ref[...] = (jnp.dot(pooled, lw_ref[...],
                            preferred_element_type=jnp.float32) + lb_ref[...])


def _finish(part, dens, exs, b, batch_p, linW, linb):
    return pl.pallas_call(
        _finish_body,
        out_shape=jax.ShapeDtypeStruct((G, OUT), jnp.float32),
    )(part, dens, exs, b.reshape(1, H), batch_p, linW, linb.reshape(1, OUT))


def kernel(x, edge_index, batch, W1, a_s1, a_d1, b1, W2, a_s2, a_d2, b2,
           linW, linb):
    del W1, a_s1, a_d1, b1
    x_p = jnp.concatenate([x, jnp.zeros((NP - N, D), jnp.float32)], axis=0)
    xpw, asrc, adst, init, exs = _prep(x_p, W2, a_s2, a_d2)
    asrc = asrc.reshape(NP)
    adst = adst.reshape(NP)

    pad = jnp.full((NW, EPT_PAD - EPT), N, jnp.int32)
    src_p = jnp.concatenate([edge_index[0].reshape(NW, EPT), pad],
                            axis=1).reshape(NW * NCH, CHUNK)
    dst_p = jnp.concatenate([edge_index[1].reshape(NW, EPT), pad],
                            axis=1).reshape(NW * NCH, CHUNK)
    eii = jnp.stack([src_p, dst_p], axis=1)
    av = jnp.concatenate([asrc, adst])

    part, dens = _edge_phase(xpw, av, eii, init)

    batch_p = jnp.concatenate(
        [batch, jnp.full((NP - N,), G, jnp.int32)]).reshape(1, NP)
    return _finish(part, dens.reshape(NW, NP), exs, b2, batch_p, linW, linb)
